# baseline (device time: 830968 ns/iter reference)
import jax
import jax.numpy as jnp
from jax import lax
from jax.experimental import pallas as pl
from jax.experimental.pallas import tpu as pltpu

N_DEV = 8
M = 4096
N = 8192
HALF = N // 2
CHUNK = M // N_DEV
TILE_N = 2048
N_TILES = HALF // TILE_N


def _body(x_ref, w_ref, out_ref,
          send_r, send_l, recv_r, recv_l, acc_r, acc_l,
          amax_buf,
          send_sems_r, recv_sems_r, send_sems_l, recv_sems_l,
          store_sems_r, store_sems_l, amax_send_sems, amax_recv_sems,
          credit_r, credit_l):
    d = lax.axis_index("i")
    right = lax.rem(d + 1, N_DEV)
    left = lax.rem(d + N_DEV - 1, N_DEV)

    barrier_sem = pltpu.get_barrier_semaphore()
    for nbr in (left, right):
        pl.semaphore_signal(barrier_sem, inc=1, device_id=(nbr,),
                            device_id_type=pl.DeviceIdType.MESH)
    pl.semaphore_wait(barrier_sem, 2)

    def partial_into(acc, c, col0):
        xc = x_ref[pl.ds(c * CHUNK, CHUNK), :]
        for j in range(N_TILES):
            acc[:, j * TILE_N:(j + 1) * TILE_N] = jnp.dot(
                xc, w_ref[:, col0 + j * TILE_N:col0 + (j + 1) * TILE_N],
                preferred_element_type=jnp.float32)

    def rdma_pair(src_r, src_l, slot):
        r = pltpu.make_async_remote_copy(
            src_ref=src_r, dst_ref=recv_r.at[slot],
            send_sem=send_sems_r.at[slot], recv_sem=recv_sems_r.at[slot],
            device_id=(right,), device_id_type=pl.DeviceIdType.MESH)
        l = pltpu.make_async_remote_copy(
            src_ref=src_l, dst_ref=recv_l.at[slot],
            send_sem=send_sems_l.at[slot], recv_sem=recv_sems_l.at[slot],
            device_id=(left,), device_id_type=pl.DeviceIdType.MESH)
        return r, l

    def give_credits():
        pl.semaphore_signal(credit_r, inc=1, device_id=(left,),
                            device_id_type=pl.DeviceIdType.MESH)
        pl.semaphore_signal(credit_l, inc=1, device_id=(right,),
                            device_id_type=pl.DeviceIdType.MESH)

    def wait_credits():
        pl.semaphore_wait(credit_r, 1)
        pl.semaphore_wait(credit_l, 1)

    partial_into(acc_r, d, 0)
    partial_into(acc_l, d, HALF)
    send_r[:, :] = acc_r[:, :].astype(jnp.bfloat16)
    send_l[:, :] = acc_l[:, :].astype(jnp.bfloat16)

    def rs_step(s, slot):
        rr, rl = rdma_pair(send_r, send_l, slot)
        rr.start()
        rl.start()
        cr = lax.rem(d + (2 * N_DEV - 1 - s), N_DEV)
        cl = lax.rem(d + s + 1, N_DEV)
        partial_into(acc_r, cr, 0)
        partial_into(acc_l, cl, HALF)
        rr.wait()
        rl.wait()
        send_r[:, :] = (acc_r[:, :]
                        + recv_r[slot].astype(jnp.float32)).astype(jnp.bfloat16)
        send_l[:, :] = (acc_l[:, :]
                        + recv_l[slot].astype(jnp.float32)).astype(jnp.bfloat16)
        give_credits()

    rs_step(0, 0)
    rs_step(1, 1)

    def rs_body(s, _):
        wait_credits()
        rs_step(s, lax.rem(s, 2))
        return _

    lax.fori_loop(2, N_DEV - 1, rs_body, None)


    my_max = jnp.maximum(jnp.max(jnp.abs(send_r[:, :].astype(jnp.float32))),
                         jnp.max(jnp.abs(send_l[:, :].astype(jnp.float32))))
    amax_buf[N_DEV - 1] = jnp.full((8, 128), my_max, jnp.float32)

    def amax_body(h, _):
        am = pltpu.make_async_remote_copy(
            src_ref=amax_buf.at[N_DEV - 1],
            dst_ref=amax_buf.at[h],
            send_sem=amax_send_sems.at[h],
            recv_sem=amax_recv_sems.at[h],
            device_id=(right,), device_id_type=pl.DeviceIdType.MESH)
        am.start()
        am.wait()
        amax_buf[N_DEV - 1] = jnp.maximum(amax_buf[N_DEV - 1],
                                          amax_buf[h])
        return _

    lax.fori_loop(0, N_DEV - 1, amax_body, None)
    scale = jnp.max(amax_buf[N_DEV - 1]) / 448.0

    def dequant_into(stg, chunk_bf16):
        for j in range(N_TILES):
            js = slice(j * TILE_N, (j + 1) * TILE_N)
            z = chunk_bf16[:, js].astype(jnp.float32) / scale
            a = jnp.abs(z)
            u = jax.lax.bitcast_convert_type(a, jnp.int32)
            r = (u + 0x7FFFF + ((u >> 20) & 1)) & ~0xFFFFF
            an = jax.lax.bitcast_convert_type(r, jnp.float32)
            asub = jnp.round(a * 512.0) * (1.0 / 512.0)
            snapped = jnp.minimum(jnp.where(a >= 2.0 ** -6, an, asub), 448.0)
            stg[:, js] = jnp.where(z < 0, -snapped, snapped) * scale

    def store(stg, g, col0, sem):
        cp = pltpu.make_async_copy(
            stg, out_ref.at[pl.ds(g * CHUNK, CHUNK), pl.ds(col0, HALF)], sem)
        cp.start()
        return cp

    wait_credits()
    rr0, rl0 = rdma_pair(send_r, send_l, 1)
    rr0.start()
    rl0.start()
    dequant_into(acc_r, send_r)
    dequant_into(acc_l, send_l)
    store(acc_r, lax.rem(d + 1, N_DEV), 0, store_sems_r.at[0])
    store(acc_l, lax.rem(d + N_DEV - 1, N_DEV), HALF, store_sems_l.at[0])
    rr0.wait()
    rl0.wait()

    def ag_body(t, _):
        slot = lax.rem(t + 1, 2)
        wait_credits()
        rr, rl = rdma_pair(recv_r.at[1 - slot], recv_l.at[1 - slot], slot)
        rr.start()
        rl.start()
        g_r = lax.rem(d + (2 * N_DEV - t) + 1, N_DEV)
        g_l = lax.rem(d + t - 1, N_DEV)
        pltpu.make_async_copy(
            acc_r, out_ref.at[pl.ds(g_r * CHUNK, CHUNK), pl.ds(0, HALF)],
            store_sems_r.at[slot]).wait()
        pltpu.make_async_copy(
            acc_l, out_ref.at[pl.ds(g_l * CHUNK, CHUNK), pl.ds(HALF, HALF)],
            store_sems_l.at[slot]).wait()
        dequant_into(acc_r, recv_r.at[1 - slot])
        dequant_into(acc_l, recv_l.at[1 - slot])
        store(acc_r, g_r, 0, store_sems_r.at[1 - slot])
        store(acc_l, g_l, HALF, store_sems_l.at[1 - slot])
        rr.wait()
        rl.wait()
        give_credits()
        return _

    lax.fori_loop(1, N_DEV - 1, ag_body, None)

    g_r = lax.rem(d + (2 * N_DEV - 6), N_DEV)
    g_l = lax.rem(d + 6, N_DEV)
    pltpu.make_async_copy(
        acc_r, out_ref.at[pl.ds(g_r * CHUNK, CHUNK), pl.ds(0, HALF)],
        store_sems_r.at[0]).wait()
    pltpu.make_async_copy(
        acc_l, out_ref.at[pl.ds(g_l * CHUNK, CHUNK), pl.ds(HALF, HALF)],
        store_sems_l.at[0]).wait()
    dequant_into(acc_r, recv_r.at[1])
    dequant_into(acc_l, recv_l.at[1])
    store(acc_r, g_r, 0, store_sems_r.at[1]).wait()
    store(acc_l, g_l, HALF, store_sems_l.at[1]).wait()
    give_credits()
    pl.semaphore_wait(credit_r, 2)
    pl.semaphore_wait(credit_l, 2)


def kernel(x, w_mat):
    x = x.astype(jnp.bfloat16)
    w_mat = w_mat.astype(jnp.bfloat16)
    return pl.pallas_call(
        _body,
        out_shape=jax.ShapeDtypeStruct((M, N), jnp.float32),
        in_specs=[
            pl.BlockSpec(memory_space=pltpu.VMEM),
            pl.BlockSpec(memory_space=pltpu.VMEM),
        ],
        out_specs=pl.BlockSpec(memory_space=pl.ANY),
        scratch_shapes=[
            pltpu.VMEM((CHUNK, HALF), jnp.bfloat16),
            pltpu.VMEM((CHUNK, HALF), jnp.bfloat16),
            pltpu.VMEM((2, CHUNK, HALF), jnp.bfloat16),
            pltpu.VMEM((2, CHUNK, HALF), jnp.bfloat16),
            pltpu.VMEM((CHUNK, HALF), jnp.float32),
            pltpu.VMEM((CHUNK, HALF), jnp.float32),
            pltpu.VMEM((N_DEV, 8, 128), jnp.float32),
            pltpu.SemaphoreType.DMA((2,)),
            pltpu.SemaphoreType.DMA((2,)),
            pltpu.SemaphoreType.DMA((2,)),
            pltpu.SemaphoreType.DMA((2,)),
            pltpu.SemaphoreType.DMA((2,)),
            pltpu.SemaphoreType.DMA((2,)),
            pltpu.SemaphoreType.DMA((N_DEV - 1,)),
            pltpu.SemaphoreType.DMA((N_DEV - 1,)),
            pltpu.SemaphoreType.REGULAR,
            pltpu.SemaphoreType.REGULAR,
        ],
        compiler_params=pltpu.CompilerParams(
            collective_id=0, vmem_limit_bytes=128 * 1024 * 1024),
    )(x, w_mat)
